# baseline (speedup 1.0000x reference)
import functools

import jax
import jax.numpy as jnp
from jax import lax
from jax.experimental import pallas as pl
from jax.experimental.pallas import tpu as pltpu
from jax.experimental.pallas import tpu_sc as plsc

_ROWS = 128
_COLS = 32768
_NC = 2
_NS = 16
_NW = _NC * _NS
_RPW = 3
_SC_ROWS = _NW * _RPW
_TC_ROWS = _ROWS - _SC_ROWS
_L = 16
_UNROLL = 8
_NIT = _COLS // (_L * _UNROLL)


def _permute(x, perm):
    dn = lax.GatherDimensionNumbers(
        offset_dims=(), collapsed_slice_dims=(0,), start_index_map=(0,)
    )
    return lax.gather(
        x, perm[:, None], dn, slice_sizes=(1,),
        mode=lax.GatherScatterMode.PROMISE_IN_BOUNDS,
    )


def _merge(m, e, pm, pe):
    better = (pm > m) | ((pm == m) & (pe < e))
    return jnp.where(better, pm, m), jnp.where(better, pe, e)


_QN = 4
_QW = _COLS // _QN
_QIT = _NIT // _QN

_INIT_CARRY_NEG = None


_GPI = 8
_EPI = _GPI * _UNROLL * _L
_IPQ = _QW // _EPI


def _scan_quarter(buf, q, carry):
    def step(it, c):
        ms = list(c[:_UNROLL])
        js = list(c[_UNROLL:-1])
        itv = c[-1]
        base = it * _EPI
        for g in range(_GPI):
            jg = itv + g
            for a in range(_UNROLL):
                v = buf[pl.ds(base + g * (_UNROLL * _L) + a * _L, _L)]
                gt = v > ms[a]
                ms[a] = jnp.where(gt, v, ms[a])
                js[a] = jnp.where(gt, jg, js[a])
        return tuple(ms) + tuple(js) + (itv + _GPI,)

    return lax.fori_loop(q * _IPQ, (q + 1) * _IPQ, step, carry)


def _finish_row(carry):
    iota = lax.iota(jnp.int32, _L)
    ms = carry[:_UNROLL]
    js = carry[_UNROLL:-1]

    m, e = ms[0], js[0] * (_UNROLL * _L) + iota
    for a in range(1, _UNROLL):
        m, e = _merge(m, e, ms[a], js[a] * (_UNROLL * _L) + (iota + a * _L))

    km = m
    for s in (8, 4, 2, 1):
        km = jnp.maximum(km, _permute(km, iota ^ s))
    cand = jnp.where(m == km, e, jnp.int32(_COLS))
    for s in (8, 4, 2, 1):
        cand = jnp.minimum(cand, _permute(cand, iota ^ s))
    return cand


@functools.partial(
    pl.kernel,
    out_type=jax.ShapeDtypeStruct((_NW, _L), jnp.int32),
    mesh=plsc.VectorSubcoreMesh(core_axis_name="c", subcore_axis_name="s"),
    scratch_types=[
        pltpu.VMEM((_COLS,), jnp.float32),
        pltpu.VMEM((_COLS,), jnp.float32),
        pltpu.VMEM((_L,), jnp.int32),
    ] + [pltpu.SemaphoreType.DMA] * (2 * _QN),
)
def _argmax_sc(x_hbm, out_hbm, buf0, buf1, outv,
               s00, s01, s02, s03, s10, s11, s12, s13):
    wid = lax.axis_index("c") * _NS + lax.axis_index("s")
    row0 = wid * _RPW
    bufs = (buf0, buf1)
    sems = ((s00, s01, s02, s03), (s10, s11, s12, s13))
    iota = lax.iota(jnp.int32, _L)

    def issue(row, q, b):
        return pltpu.async_copy(
            x_hbm.at[row, pl.ds(q * _QW, _QW)],
            bufs[b].at[pl.ds(q * _QW, _QW)],
            sems[b][q],
        )

    handles = {}
    for q in range(_QN):
        handles[(0, q)] = issue(row0, q, 0)
    for q in range(_QN):
        handles[(1, q)] = issue(row0 + 1, q, 1)

    neg = jnp.full((_L,), -jnp.inf, dtype=jnp.float32)
    zero = jnp.zeros((_L,), dtype=jnp.int32)
    acc = jnp.zeros((_L,), dtype=jnp.int32)
    for k in range(_RPW):
        b = k % 2
        carry = (neg,) * _UNROLL + (zero,) * _UNROLL + (zero,)
        for q in range(_QN):
            handles[(k, q)].wait()
            carry = _scan_quarter(bufs[b], q, carry)
            if k + 2 < _RPW:
                handles[(k + 2, q)] = issue(row0 + k + 2, q, b)
        r = _finish_row(carry)
        acc = jnp.where(iota == k, r, acc)
    outv[...] = acc
    pltpu.sync_copy(outv, out_hbm.at[wid])


_BR = 8
_W = 512
_NCH = _COLS // _W


def _tc_body(x_ref, out_ref):
    iota_w = lax.broadcasted_iota(jnp.int32, (_BR, _W), 1)

    def step(c, carry):
        m, j = carry
        v = x_ref[:, pl.ds(c * _W, _W)]
        gt = v > m
        return jnp.where(gt, v, m), jnp.where(gt, c, j)

    m0 = jnp.full((_BR, _W), -jnp.inf, jnp.float32)
    j0 = jnp.zeros((_BR, _W), jnp.int32)
    m, j = lax.fori_loop(0, _NCH, step, (m0, j0))

    elem = j * _W + iota_w
    rowmax = jnp.max(m, axis=1, keepdims=True)
    cand = jnp.where(m == rowmax, elem, jnp.int32(_COLS))
    out_ref[0, 0, :] = jnp.min(cand, axis=1)


_tc_argmax = pl.pallas_call(
    _tc_body,
    grid=(_TC_ROWS // _BR,),
    in_specs=[pl.BlockSpec((_BR, _COLS), lambda i: (i + _SC_ROWS // _BR, 0))],
    out_specs=pl.BlockSpec((1, 1, _BR), lambda i: (i, 0, 0)),
    out_shape=jax.ShapeDtypeStruct((_TC_ROWS // _BR, 1, _BR), jnp.int32),
)


def kernel(x):
    sc_out = _argmax_sc(x)
    tc_out = _tc_argmax(x)
    return jnp.concatenate(
        [sc_out[:, :_RPW].reshape(_SC_ROWS), tc_out.reshape(_TC_ROWS)]
    )


# speedup vs baseline: 1.5073x; 1.5073x over previous
import functools

import jax
import jax.numpy as jnp
from jax import lax
from jax.experimental import pallas as pl
from jax.experimental.pallas import tpu as pltpu
from jax.experimental.pallas import tpu_sc as plsc

_ROWS = 128
_COLS = 32768
_NC = 2
_NS = 16
_NW = _NC * _NS
_RPW = 3
_SC_ROWS = _NW * _RPW
_TC_ROWS = _ROWS - _SC_ROWS
_L = 16
_UNROLL = 8
_NIT = _COLS // (_L * _UNROLL)


def _permute(x, perm):
    dn = lax.GatherDimensionNumbers(
        offset_dims=(), collapsed_slice_dims=(0,), start_index_map=(0,)
    )
    return lax.gather(
        x, perm[:, None], dn, slice_sizes=(1,),
        mode=lax.GatherScatterMode.PROMISE_IN_BOUNDS,
    )


def _merge(m, e, pm, pe):
    better = (pm > m) | ((pm == m) & (pe < e))
    return jnp.where(better, pm, m), jnp.where(better, pe, e)


_QN = 4
_QW = _COLS // _QN
_QIT = _NIT // _QN

_INIT_CARRY_NEG = None


def _scan_quarter(buf, q, carry):
    def step(it, c):
        ms = c[:_UNROLL]
        js = c[_UNROLL:-1]
        itv = c[-1]
        new_ms, new_js = [], []
        base = it * _UNROLL
        for a in range(_UNROLL):
            v = buf[pl.ds((base + a) * _L, _L)]
            gt = v > ms[a]
            new_ms.append(jnp.where(gt, v, ms[a]))
            new_js.append(jnp.where(gt, itv, js[a]))
        return tuple(new_ms) + tuple(new_js) + (itv + 1,)

    return lax.fori_loop(q * _QIT, (q + 1) * _QIT, step, carry)


def _finish_row(carry):
    iota = lax.iota(jnp.int32, _L)
    ms = carry[:_UNROLL]
    js = carry[_UNROLL:-1]

    m, e = ms[0], js[0] * (_UNROLL * _L) + iota
    for a in range(1, _UNROLL):
        m, e = _merge(m, e, ms[a], js[a] * (_UNROLL * _L) + (iota + a * _L))

    km = m
    for s in (8, 4, 2, 1):
        km = jnp.maximum(km, _permute(km, iota ^ s))
    cand = jnp.where(m == km, e, jnp.int32(_COLS))
    for s in (8, 4, 2, 1):
        cand = jnp.minimum(cand, _permute(cand, iota ^ s))
    return cand


@functools.partial(
    pl.kernel,
    out_type=jax.ShapeDtypeStruct((_NW, _L), jnp.int32),
    mesh=plsc.VectorSubcoreMesh(core_axis_name="c", subcore_axis_name="s"),
    scratch_types=[
        pltpu.VMEM((_COLS,), jnp.float32),
        pltpu.VMEM((_COLS,), jnp.float32),
        pltpu.VMEM((_L,), jnp.int32),
    ] + [pltpu.SemaphoreType.DMA] * (2 * _QN),
)
def _argmax_sc(x_hbm, out_hbm, buf0, buf1, outv,
               s00, s01, s02, s03, s10, s11, s12, s13):
    wid = lax.axis_index("c") * _NS + lax.axis_index("s")
    row0 = wid * _RPW
    bufs = (buf0, buf1)
    sems = ((s00, s01, s02, s03), (s10, s11, s12, s13))
    iota = lax.iota(jnp.int32, _L)

    def issue(row, q, b):
        return pltpu.async_copy(
            x_hbm.at[row, pl.ds(q * _QW, _QW)],
            bufs[b].at[pl.ds(q * _QW, _QW)],
            sems[b][q],
        )

    handles = {}
    for q in range(_QN):
        handles[(0, q)] = issue(row0, q, 0)
    for q in range(_QN):
        handles[(1, q)] = issue(row0 + 1, q, 1)

    neg = jnp.full((_L,), -jnp.inf, dtype=jnp.float32)
    zero = jnp.zeros((_L,), dtype=jnp.int32)
    acc = jnp.zeros((_L,), dtype=jnp.int32)
    for k in range(_RPW):
        b = k % 2
        carry = (neg,) * _UNROLL + (zero,) * _UNROLL + (zero,)
        for q in range(_QN):
            handles[(k, q)].wait()
            carry = _scan_quarter(bufs[b], q, carry)
            if k + 2 < _RPW:
                handles[(k + 2, q)] = issue(row0 + k + 2, q, b)
        r = _finish_row(carry)
        acc = jnp.where(iota == k, r, acc)
    outv[...] = acc
    pltpu.sync_copy(outv, out_hbm.at[wid])


_BR = 8
_W = 512
_NCH = _COLS // _W


def _tc_body(x_ref, out_ref):
    iota_w = lax.broadcasted_iota(jnp.int32, (_BR, _W), 1)

    def step(c, carry):
        m, j = carry
        v = x_ref[:, pl.ds(c * _W, _W)]
        gt = v > m
        return jnp.where(gt, v, m), jnp.where(gt, c, j)

    m0 = jnp.full((_BR, _W), -jnp.inf, jnp.float32)
    j0 = jnp.zeros((_BR, _W), jnp.int32)
    m, j = lax.fori_loop(0, _NCH, step, (m0, j0))

    elem = j * _W + iota_w
    rowmax = jnp.max(m, axis=1, keepdims=True)
    cand = jnp.where(m == rowmax, elem, jnp.int32(_COLS))
    out_ref[0, 0, :] = jnp.min(cand, axis=1)


_tc_argmax = pl.pallas_call(
    _tc_body,
    grid=(_TC_ROWS // _BR,),
    in_specs=[pl.BlockSpec((_BR, _COLS), lambda i: (i + _SC_ROWS // _BR, 0))],
    out_specs=pl.BlockSpec((1, 1, _BR), lambda i: (i, 0, 0)),
    out_shape=jax.ShapeDtypeStruct((_TC_ROWS // _BR, 1, _BR), jnp.int32),
)


def kernel(x):
    sc_out = _argmax_sc(x)
    tc_out = _tc_argmax(x)
    return jnp.concatenate(
        [sc_out[:, :_RPW].reshape(_SC_ROWS), tc_out.reshape(_TC_ROWS)]
    )
